# baseline (speedup 1.0000x reference)
import functools

import jax
import jax.numpy as jnp
from jax import lax
from jax.experimental import pallas as pl
from jax.experimental.pallas import tpu as pltpu
from jax.experimental.pallas import tpu_sc as plsc

B, L, D, C = 4096, 50, 64, 4
NC, NS = 2, 16
NW = NC * NS
BPW = B // NW
CH = 2
NCHUNK = BPW // CH
LANES = 16
QD = D // LANES

_mesh = plsc.VectorSubcoreMesh(
    core_axis_name="c", subcore_axis_name="s", num_cores=NC, num_subcores=NS)


@functools.partial(
    pl.kernel,
    out_type=jax.ShapeDtypeStruct((B, D), jnp.float32),
    mesh=_mesh,
    scratch_types=[
        pltpu.VMEM((NCHUNK, CH * L), jnp.int32),
        pltpu.VMEM((CH * L, D), jnp.float32),
        pltpu.VMEM((CH * L, D), jnp.float32),
        pltpu.VMEM((BPW, D), jnp.float32),
        pltpu.SemaphoreType.DMA,
        pltpu.SemaphoreType.DMA,
    ],
    compiler_params=pltpu.CompilerParams(use_tc_tiling_on_sc=False),
)
def _pool_sum(text_hbm, table_hbm, out_hbm, idx_v, rows0, rows1, pooled_v,
              sem0, sem1):
    wid = lax.axis_index("s") * NC + lax.axis_index("c")
    base = wid * BPW
    pltpu.sync_copy(text_hbm.at[pl.ds(wid * NCHUNK, NCHUNK)], idx_v)
    bufs = (rows0, rows1)
    sems = (sem0, sem1)

    def start(c):
        return pltpu.async_copy(
            table_hbm.at[idx_v.at[c]], bufs[c % 2], sems[c % 2])

    pending = start(0)
    for c in range(NCHUNK):
        nxt = start(c + 1) if c + 1 < NCHUNK else None
        pending.wait()
        rows = bufs[c % 2]
        for b in range(CH):
            def body(j, acc, _rows=rows, _b=b):
                return tuple(acc[q] + _rows[_b * L + j, pl.ds(q * LANES, LANES)]
                             for q in range(QD))
            acc = lax.fori_loop(
                1, L, body,
                tuple(rows[b * L, pl.ds(q * LANES, LANES)] for q in range(QD)))
            for q in range(QD):
                pooled_v[c * CH + b, pl.ds(q * LANES, LANES)] = acc[q]
        pending = nxt
    pltpu.sync_copy(pooled_v, out_hbm.at[pl.ds(base, BPW)])


def _linear_body(p_ref, w_ref, b_ref, o_ref):
    o_ref[...] = lax.dot_general(
        p_ref[...], w_ref[...], (((1,), (1,)), ((), ())),
        precision=lax.Precision.HIGHEST,
        preferred_element_type=jnp.float32) + b_ref[...]


def kernel(text, table, Wfc, bfc):
    pooled = _pool_sum(text.reshape(NW * NCHUNK, CH * L), table)
    w_scaled = Wfc * (1.0 / L)
    out = pl.pallas_call(
        _linear_body,
        out_shape=jax.ShapeDtypeStruct((B, C), jnp.float32),
    )(pooled, w_scaled, bfc.reshape(1, C))
    return out
